# Initial kernel scaffold; baseline (speedup 1.0000x reference)
import jax
import jax.numpy as jnp
from jax.experimental import pallas as pl


def kernel(x, edge_index, W_e1, b_e1, W_e2, b_e2, W_n1, b_n1, W_n2, b_n2, W_g, b_g):
    raise NotImplementedError("write your pallas kernel here")


# speedup vs baseline: 2.4435x; 2.4435x over previous
import functools

import jax
import jax.numpy as jnp
from jax import lax
from jax.experimental import pallas as pl
from jax.experimental.pallas import tpu as pltpu
from jax.experimental.pallas import tpu_sc as plsc

N = 10000
E = 320000
D = 128

NC = 2
NS = 16
NW = NC * NS
PER_TILE = E // NW
CH = 80
NH = 5000
TR = NH + 8

_F32 = jnp.float32
_HI = lax.Precision.HIGHEST


def _dot(a, b):
    return jnp.dot(a, b, precision=_HI, preferred_element_type=_F32)


def _tc_pq(x0, w1a, w1b, b1):
    def body(x_ref, wa_ref, wb_ref, b_ref, p_ref, q_ref):
        xv = x_ref[...]
        p_ref[...] = _dot(xv, wa_ref[...]) + b_ref[...]
        q_ref[...] = _dot(xv, wb_ref[...])

    return pl.pallas_call(
        body,
        out_shape=(jax.ShapeDtypeStruct((N, D), _F32),
                   jax.ShapeDtypeStruct((N, D), _F32)),
    )(x0, w1a, w1b, b1.reshape(1, D))


def _split_idx(idxv, idxa, idxb):
    @pl.loop(0, CH, step=16)
    def _(j):
        v = idxv[pl.ds(j, 16)]
        lo = v < NH
        idxa[pl.ds(j, 16)] = jnp.where(lo, v, NH)
        idxb[pl.ds(j, 16)] = jnp.where(lo, NH, v - NH)


def _sc_gather(p, q, row, col, ztr, ones_c):
    mesh = plsc.VectorSubcoreMesh(core_axis_name="c", subcore_axis_name="s")

    @functools.partial(
        pl.kernel,
        mesh=mesh,
        out_type=(jax.ShapeDtypeStruct((E, D), _F32),
                  jax.ShapeDtypeStruct((E, D), _F32),
                  jax.ShapeDtypeStruct((NC * N, D), _F32)),
        scratch_types=[
            pltpu.VMEM((CH,), jnp.int32),
            pltpu.VMEM((CH,), jnp.int32),
            pltpu.VMEM((CH,), jnp.int32),
            pltpu.VMEM((CH,), jnp.int32),
            pltpu.VMEM((CH, D), _F32),
            pltpu.VMEM((CH, D), _F32),
            pltpu.VMEM((CH, D), _F32),
            pltpu.VMEM_SHARED((TR, D), _F32),
            pltpu.VMEM_SHARED((TR, D), _F32),
            pltpu.SemaphoreType.DMA,
            pltpu.SemaphoreType.DMA,
        ],
    )
    def gk(p_hbm, q_hbm, row_hbm, col_hbm, ztr_hbm, ones_hbm,
           pg_hbm, qg_hbm, cnt_hbm,
           idxr, idxc, idxa, idxb, bufp, bufq, ones_v, ca_sh, cb_sh,
           sem1, sem2):
        c = lax.axis_index("c")
        s = lax.axis_index("s")
        pltpu.sync_copy(ones_hbm, ones_v)

        @pl.when(s == 0)
        def _():
            pltpu.sync_copy(ztr_hbm, ca_sh)
            pltpu.sync_copy(ztr_hbm, cb_sh)

        plsc.subcore_barrier()

        wid = s * NC + c
        base = wid * PER_TILE

        @pl.loop(0, PER_TILE, step=CH)
        def _(off):
            b = base + off
            pltpu.sync_copy(row_hbm.at[pl.ds(b, CH)], idxr)
            pltpu.sync_copy(col_hbm.at[pl.ds(b, CH)], idxc)
            cp1 = pltpu.async_copy(p_hbm.at[idxr], bufp, sem1)
            cp2 = pltpu.async_copy(q_hbm.at[idxc], bufq, sem2)
            _split_idx(idxc, idxa, idxb)
            cp1.wait()
            cp2.wait()
            pltpu.sync_copy(bufp, pg_hbm.at[pl.ds(b, CH)])
            pltpu.sync_copy(bufq, qg_hbm.at[pl.ds(b, CH)])
            pltpu.sync_copy(ones_v, ca_sh.at[idxa], add=True)
            pltpu.sync_copy(ones_v, cb_sh.at[idxb], add=True)

        plsc.subcore_barrier()

        @pl.when(s == 0)
        def _():
            pltpu.sync_copy(ca_sh.at[pl.ds(0, NH)],
                            cnt_hbm.at[pl.ds(c * N, NH)])
            pltpu.sync_copy(cb_sh.at[pl.ds(0, NH)],
                            cnt_hbm.at[pl.ds(c * N + NH, NH)])

    return gk(p, q, row, col, ztr, ones_c)


EB = 2000


def _tc_edge(pg, qg, w2, b2):
    def body(pg_ref, qg_ref, w2_ref, b2_ref, out_ref):
        h = jnp.maximum(pg_ref[...] + qg_ref[...], 0.0)
        out_ref[...] = _dot(h, w2_ref[...]) + b2_ref[...]

    return pl.pallas_call(
        body,
        grid=(E // EB,),
        in_specs=[
            pl.BlockSpec((EB, D), lambda i: (i, 0)),
            pl.BlockSpec((EB, D), lambda i: (i, 0)),
            pl.BlockSpec((D, D), lambda i: (0, 0)),
            pl.BlockSpec((1, D), lambda i: (0, 0)),
        ],
        out_specs=pl.BlockSpec((EB, D), lambda i: (i, 0)),
        out_shape=jax.ShapeDtypeStruct((E, D), _F32),
        compiler_params=pltpu.CompilerParams(
            dimension_semantics=("parallel",)),
    )(pg, qg, w2, b2.reshape(1, D))


def _sc_segsum(ea, col, ztr):
    mesh = plsc.VectorSubcoreMesh(core_axis_name="c", subcore_axis_name="s")

    @functools.partial(
        pl.kernel,
        mesh=mesh,
        out_type=jax.ShapeDtypeStruct((NC * N, D), _F32),
        scratch_types=[
            pltpu.VMEM((CH,), jnp.int32),
            pltpu.VMEM((CH,), jnp.int32),
            pltpu.VMEM((CH,), jnp.int32),
            pltpu.VMEM((CH, D), _F32),
            pltpu.VMEM_SHARED((TR, D), _F32),
            pltpu.VMEM_SHARED((TR, D), _F32),
        ],
    )
    def sk(ea_hbm, col_hbm, ztr_hbm, part_hbm,
           idxv, idxa, idxb, buf, ta_sh, tb_sh):
        c = lax.axis_index("c")
        s = lax.axis_index("s")

        @pl.when(s == 0)
        def _():
            pltpu.sync_copy(ztr_hbm, ta_sh)
            pltpu.sync_copy(ztr_hbm, tb_sh)

        plsc.subcore_barrier()

        wid = s * NC + c
        base = wid * PER_TILE

        @pl.loop(0, PER_TILE, step=CH)
        def _(off):
            b = base + off
            pltpu.sync_copy(col_hbm.at[pl.ds(b, CH)], idxv)
            pltpu.sync_copy(ea_hbm.at[pl.ds(b, CH)], buf)
            _split_idx(idxv, idxa, idxb)
            pltpu.sync_copy(buf, ta_sh.at[idxa], add=True)
            pltpu.sync_copy(buf, tb_sh.at[idxb], add=True)

        plsc.subcore_barrier()

        @pl.when(s == 0)
        def _():
            pltpu.sync_copy(ta_sh.at[pl.ds(0, NH)],
                            part_hbm.at[pl.ds(c * N, NH)])
            pltpu.sync_copy(tb_sh.at[pl.ds(0, NH)],
                            part_hbm.at[pl.ds(c * N + NH, NH)])

    return sk(ea, col, ztr)


def _tc_node(x0, part, cnt, wn1a, wn1b, bn1, wn2, bn2, wg, bg):
    def body(x_ref, part_ref, cnt_ref, wa_ref, wb_ref, b1_ref,
             w2_ref, b2_ref, wg_ref, bg_ref, xo_ref, u_ref):
        cntv = cnt_ref[0, :, 0:1] + cnt_ref[1, :, 0:1]
        agg = (part_ref[0] + part_ref[1]) / jnp.maximum(cntv, 1.0)
        h2 = jnp.maximum(
            _dot(x_ref[...], wa_ref[...]) + _dot(agg, wb_ref[...])
            + b1_ref[...], 0.0)
        xo = _dot(h2, w2_ref[...]) + b2_ref[...]
        xo_ref[...] = xo
        u_ref[...] = _dot(jnp.sum(xo, axis=0, keepdims=True) / N,
                          wg_ref[...]) + bg_ref[...]

    return pl.pallas_call(
        body,
        out_shape=(jax.ShapeDtypeStruct((N, D), _F32),
                   jax.ShapeDtypeStruct((1, D), _F32)),
    )(x0, part, cnt, wn1a, wn1b, bn1.reshape(1, D),
      wn2, bn2.reshape(1, D), wg, bg.reshape(1, D))


def kernel(x, edge_index, W_e1, b_e1, W_e2, b_e2,
           W_n1, b_n1, W_n2, b_n2, W_g, b_g):
    x0 = x[0]
    row = edge_index[0]
    col = edge_index[1]
    ztr = jnp.zeros((TR, D), _F32)
    ones_c = jnp.ones((CH, D), _F32)

    p, q = _tc_pq(x0, W_e1[:D], W_e1[D:], b_e1)
    pg, qg, cnt2 = _sc_gather(p, q, row, col, ztr, ones_c)
    ea = _tc_edge(pg, qg, W_e2, b_e2)
    part2 = _sc_segsum(ea, col, ztr)
    part = part2.reshape(NC, N, D)
    cnt = cnt2.reshape(NC, N, D)
    x_out, u = _tc_node(x0, part, cnt, W_n1[:D], W_n1[D:], b_n1,
                        W_n2, b_n2, W_g, b_g)
    return (x_out[None], ea[None], u)


# speedup vs baseline: 2.7273x; 1.1161x over previous
import functools

import jax
import jax.numpy as jnp
from jax import lax
from jax.experimental import pallas as pl
from jax.experimental.pallas import tpu as pltpu
from jax.experimental.pallas import tpu_sc as plsc

N = 10000
E = 320000
D = 128

NC = 2
NS = 16
NW = NC * NS
PER_TILE = E // NW
CH = 80
NH = 5000
TR = NH + 8

_F32 = jnp.float32
_HI = lax.Precision.HIGHEST


def _dot(a, b):
    return jnp.dot(a, b, precision=_HI, preferred_element_type=_F32)


def _tc_pq(x0, w1a, w1b, b1):
    def body(x_ref, wa_ref, wb_ref, b_ref, p_ref, q_ref):
        xv = x_ref[...]
        p_ref[...] = _dot(xv, wa_ref[...]) + b_ref[...]
        q_ref[...] = _dot(xv, wb_ref[...])

    return pl.pallas_call(
        body,
        out_shape=(jax.ShapeDtypeStruct((N, D), _F32),
                   jax.ShapeDtypeStruct((N, D), _F32)),
    )(x0, w1a, w1b, b1.reshape(1, D))


def _split_idx(idxv, idxa, idxb, off=0):
    @pl.loop(0, CH, step=16)
    def _(j):
        v = idxv[pl.ds(off + j, 16)]
        lo = v < NH
        idxa[pl.ds(j, 16)] = jnp.where(lo, v, NH)
        idxb[pl.ds(j, 16)] = jnp.where(lo, NH, v - NH)


def _sc_gather(p, q, row, col, ztr, ones_c):
    mesh = plsc.VectorSubcoreMesh(core_axis_name="c", subcore_axis_name="s")

    @functools.partial(
        pl.kernel,
        mesh=mesh,
        out_type=(jax.ShapeDtypeStruct((E, D), _F32),
                  jax.ShapeDtypeStruct((E, D), _F32),
                  jax.ShapeDtypeStruct((NC * N, D), _F32)),
        scratch_types=[
            pltpu.VMEM((PER_TILE,), jnp.int32),
            pltpu.VMEM((CH,), jnp.int32),
            pltpu.VMEM((CH,), jnp.int32),
            pltpu.VMEM((CH,), jnp.int32),
            pltpu.VMEM((CH, D), _F32),
            pltpu.VMEM((CH, D), _F32),
            pltpu.VMEM((CH, D), _F32),
            pltpu.VMEM_SHARED((TR, D), _F32),
            pltpu.VMEM_SHARED((TR, D), _F32),
            pltpu.SemaphoreType.DMA,
            pltpu.SemaphoreType.DMA,
            pltpu.SemaphoreType.DMA,
        ],
    )
    def gk(p_hbm, q_hbm, row_hbm, col_hbm, ztr_hbm, ones_hbm,
           pg_hbm, qg_hbm, cnt_hbm,
           idxc_all, idxr, idxa, idxb,
           bufp, bufq, ones_v, ca_sh, cb_sh,
           semg0, semg1, semw):
        c = lax.axis_index("c")
        s = lax.axis_index("s")
        pltpu.sync_copy(ones_hbm, ones_v)

        @pl.when(s == 0)
        def _():
            pltpu.sync_copy(ztr_hbm, ca_sh)
            pltpu.sync_copy(ztr_hbm, cb_sh)

        wid = s * NC + c
        base = wid * PER_TILE
        pltpu.sync_copy(col_hbm.at[pl.ds(base, PER_TILE)], idxc_all)
        plsc.subcore_barrier()

        @pl.loop(0, PER_TILE, step=CH)
        def _(off):
            b = base + off
            pltpu.sync_copy(row_hbm.at[pl.ds(b, CH)], idxr)

            @pl.when(off > 0)
            def _():
                for _ in range(2):
                    pltpu.make_async_copy(bufp, pg_hbm.at[pl.ds(0, CH)],
                                          semw).wait()

            cp1 = pltpu.async_copy(p_hbm.at[idxr], bufp, semg0)
            cp2 = pltpu.async_copy(q_hbm.at[idxc_all.at[pl.ds(off, CH)]],
                                   bufq, semg1)
            _split_idx(idxc_all, idxa, idxb, off)
            cp1.wait()
            cp2.wait()
            pltpu.async_copy(bufp, pg_hbm.at[pl.ds(b, CH)], semw)
            pltpu.async_copy(bufq, qg_hbm.at[pl.ds(b, CH)], semw)
            pltpu.sync_copy(ones_v, ca_sh.at[idxa], add=True)
            pltpu.sync_copy(ones_v, cb_sh.at[idxb], add=True)

        for _ in range(2):
            pltpu.make_async_copy(bufp, pg_hbm.at[pl.ds(0, CH)], semw).wait()

        plsc.subcore_barrier()

        @pl.when(s == 0)
        def _():
            pltpu.sync_copy(ca_sh.at[pl.ds(0, NH)],
                            cnt_hbm.at[pl.ds(c * N, NH)])
            pltpu.sync_copy(cb_sh.at[pl.ds(0, NH)],
                            cnt_hbm.at[pl.ds(c * N + NH, NH)])

    return gk(p, q, row, col, ztr, ones_c)


EB = 2000


def _tc_edge(pg, qg, w2, b2):
    def body(pg_ref, qg_ref, w2_ref, b2_ref, out_ref):
        h = jnp.maximum(pg_ref[...] + qg_ref[...], 0.0)
        out_ref[...] = _dot(h, w2_ref[...]) + b2_ref[...]

    return pl.pallas_call(
        body,
        grid=(E // EB,),
        in_specs=[
            pl.BlockSpec((EB, D), lambda i: (i, 0)),
            pl.BlockSpec((EB, D), lambda i: (i, 0)),
            pl.BlockSpec((D, D), lambda i: (0, 0)),
            pl.BlockSpec((1, D), lambda i: (0, 0)),
        ],
        out_specs=pl.BlockSpec((EB, D), lambda i: (i, 0)),
        out_shape=jax.ShapeDtypeStruct((E, D), _F32),
        compiler_params=pltpu.CompilerParams(
            dimension_semantics=("parallel",)),
    )(pg, qg, w2, b2.reshape(1, D))


def _sc_segsum(ea, col, ztr):
    mesh = plsc.VectorSubcoreMesh(core_axis_name="c", subcore_axis_name="s")

    @functools.partial(
        pl.kernel,
        mesh=mesh,
        out_type=jax.ShapeDtypeStruct((NC * N, D), _F32),
        scratch_types=[
            pltpu.VMEM((CH,), jnp.int32),
            pltpu.VMEM((CH,), jnp.int32),
            pltpu.VMEM((CH,), jnp.int32),
            pltpu.VMEM((CH, D), _F32),
            pltpu.VMEM_SHARED((TR, D), _F32),
            pltpu.VMEM_SHARED((TR, D), _F32),
        ],
    )
    def sk(ea_hbm, col_hbm, ztr_hbm, part_hbm,
           idxv, idxa, idxb, buf, ta_sh, tb_sh):
        c = lax.axis_index("c")
        s = lax.axis_index("s")

        @pl.when(s == 0)
        def _():
            pltpu.sync_copy(ztr_hbm, ta_sh)
            pltpu.sync_copy(ztr_hbm, tb_sh)

        plsc.subcore_barrier()

        wid = s * NC + c
        base = wid * PER_TILE

        @pl.loop(0, PER_TILE, step=CH)
        def _(off):
            b = base + off
            pltpu.sync_copy(col_hbm.at[pl.ds(b, CH)], idxv)
            pltpu.sync_copy(ea_hbm.at[pl.ds(b, CH)], buf)
            _split_idx(idxv, idxa, idxb)
            pltpu.sync_copy(buf, ta_sh.at[idxa], add=True)
            pltpu.sync_copy(buf, tb_sh.at[idxb], add=True)

        plsc.subcore_barrier()

        @pl.when(s == 0)
        def _():
            pltpu.sync_copy(ta_sh.at[pl.ds(0, NH)],
                            part_hbm.at[pl.ds(c * N, NH)])
            pltpu.sync_copy(tb_sh.at[pl.ds(0, NH)],
                            part_hbm.at[pl.ds(c * N + NH, NH)])

    return sk(ea, col, ztr)


def _tc_node(x0, part, cnt, wn1a, wn1b, bn1, wn2, bn2, wg, bg):
    def body(x_ref, part_ref, cnt_ref, wa_ref, wb_ref, b1_ref,
             w2_ref, b2_ref, wg_ref, bg_ref, xo_ref, u_ref):
        cntv = cnt_ref[0, :, 0:1] + cnt_ref[1, :, 0:1]
        agg = (part_ref[0] + part_ref[1]) / jnp.maximum(cntv, 1.0)
        h2 = jnp.maximum(
            _dot(x_ref[...], wa_ref[...]) + _dot(agg, wb_ref[...])
            + b1_ref[...], 0.0)
        xo = _dot(h2, w2_ref[...]) + b2_ref[...]
        xo_ref[...] = xo
        u_ref[...] = _dot(jnp.sum(xo, axis=0, keepdims=True) / N,
                          wg_ref[...]) + bg_ref[...]

    return pl.pallas_call(
        body,
        out_shape=(jax.ShapeDtypeStruct((N, D), _F32),
                   jax.ShapeDtypeStruct((1, D), _F32)),
    )(x0, part, cnt, wn1a, wn1b, bn1.reshape(1, D),
      wn2, bn2.reshape(1, D), wg, bg.reshape(1, D))


def kernel(x, edge_index, W_e1, b_e1, W_e2, b_e2,
           W_n1, b_n1, W_n2, b_n2, W_g, b_g):
    x0 = x[0]
    row = edge_index[0]
    col = edge_index[1]
    ztr = jnp.zeros((TR, D), _F32)
    ones_c = jnp.ones((CH, D), _F32)

    p, q = _tc_pq(x0, W_e1[:D], W_e1[D:], b_e1)
    pg, qg, cnt2 = _sc_gather(p, q, row, col, ztr, ones_c)
    ea = _tc_edge(pg, qg, W_e2, b_e2)
    part2 = _sc_segsum(ea, col, ztr)
    part = part2.reshape(NC, N, D)
    cnt = cnt2.reshape(NC, N, D)
    x_out, u = _tc_node(x0, part, cnt, W_n1[:D], W_n1[D:], b_n1,
                        W_n2, b_n2, W_g, b_g)
    return (x_out[None], ea[None], u)


# speedup vs baseline: 2.8553x; 1.0469x over previous
import functools

import jax
import jax.numpy as jnp
from jax import lax
from jax.experimental import pallas as pl
from jax.experimental.pallas import tpu as pltpu
from jax.experimental.pallas import tpu_sc as plsc

N = 10000
E = 320000
D = 128

NC = 2
NS = 16
NW = NC * NS
PER_TILE = E // NW
CH = 80
NH = 5000
TR = NH + 8

_F32 = jnp.float32
_HI = lax.Precision.HIGHEST


def _dot(a, b):
    return jnp.dot(a, b, precision=_HI, preferred_element_type=_F32)


def _tc_pq(x0, w1a, w1b, b1):
    def body(x_ref, wa_ref, wb_ref, b_ref, p_ref, q_ref):
        xv = x_ref[...]
        p_ref[...] = _dot(xv, wa_ref[...]) + b_ref[...]
        q_ref[...] = _dot(xv, wb_ref[...])

    return pl.pallas_call(
        body,
        out_shape=(jax.ShapeDtypeStruct((N, D), _F32),
                   jax.ShapeDtypeStruct((N, D), _F32)),
    )(x0, w1a, w1b, b1.reshape(1, D))


def _split_idx(idxv, idxa, idxb, off=0):
    @pl.loop(0, CH, step=16)
    def _(j):
        v = idxv[pl.ds(off + j, 16)]
        lo = v < NH
        idxa[pl.ds(j, 16)] = jnp.where(lo, v, NH)
        idxb[pl.ds(j, 16)] = jnp.where(lo, NH, v - NH)


def _sc_gather(p, q, row, col, ztr, ones_c):
    mesh = plsc.VectorSubcoreMesh(core_axis_name="c", subcore_axis_name="s")

    @functools.partial(
        pl.kernel,
        mesh=mesh,
        out_type=(jax.ShapeDtypeStruct((E, D), _F32),
                  jax.ShapeDtypeStruct((E, D), _F32),
                  jax.ShapeDtypeStruct((NC * N, D), _F32)),
        scratch_types=[
            pltpu.VMEM((PER_TILE,), jnp.int32),
            pltpu.VMEM((CH,), jnp.int32),
            pltpu.VMEM((CH,), jnp.int32),
            pltpu.VMEM((CH,), jnp.int32),
            pltpu.VMEM((CH, D), _F32),
            pltpu.VMEM((CH, D), _F32),
            pltpu.VMEM((CH, D), _F32),
            pltpu.VMEM_SHARED((TR, D), _F32),
            pltpu.VMEM_SHARED((TR, D), _F32),
            pltpu.SemaphoreType.DMA,
            pltpu.SemaphoreType.DMA,
            pltpu.SemaphoreType.DMA,
        ],
    )
    def gk(p_hbm, q_hbm, row_hbm, col_hbm, ztr_hbm, ones_hbm,
           pg_hbm, qg_hbm, cnt_hbm,
           idxc_all, idxr, idxa, idxb,
           bufp, bufq, ones_v, ca_sh, cb_sh,
           semg0, semg1, semw):
        c = lax.axis_index("c")
        s = lax.axis_index("s")
        pltpu.sync_copy(ones_hbm, ones_v)

        @pl.when(s == 0)
        def _():
            pltpu.sync_copy(ztr_hbm, ca_sh)
            pltpu.sync_copy(ztr_hbm, cb_sh)

        wid = s * NC + c
        base = wid * PER_TILE
        pltpu.sync_copy(col_hbm.at[pl.ds(base, PER_TILE)], idxc_all)
        plsc.subcore_barrier()

        @pl.loop(0, PER_TILE, step=CH)
        def _(off):
            b = base + off
            pltpu.sync_copy(row_hbm.at[pl.ds(b, CH)], idxr)

            @pl.when(off > 0)
            def _():
                for _ in range(2):
                    pltpu.make_async_copy(bufp, pg_hbm.at[pl.ds(0, CH)],
                                          semw).wait()

            cp1 = pltpu.async_copy(p_hbm.at[idxr], bufp, semg0)
            cp2 = pltpu.async_copy(q_hbm.at[idxc_all.at[pl.ds(off, CH)]],
                                   bufq, semg1)
            _split_idx(idxc_all, idxa, idxb, off)
            cp1.wait()
            cp2.wait()
            pltpu.async_copy(bufp, pg_hbm.at[pl.ds(b, CH)], semw)
            pltpu.async_copy(bufq, qg_hbm.at[pl.ds(b, CH)], semw)
            pltpu.sync_copy(ones_v, ca_sh.at[idxa], add=True)
            pltpu.sync_copy(ones_v, cb_sh.at[idxb], add=True)

        for _ in range(2):
            pltpu.make_async_copy(bufp, pg_hbm.at[pl.ds(0, CH)], semw).wait()

        plsc.subcore_barrier()

        @pl.when(s == 0)
        def _():
            pltpu.sync_copy(ca_sh.at[pl.ds(0, NH)],
                            cnt_hbm.at[pl.ds(c * N, NH)])
            pltpu.sync_copy(cb_sh.at[pl.ds(0, NH)],
                            cnt_hbm.at[pl.ds(c * N + NH, NH)])

    return gk(p, q, row, col, ztr, ones_c)


EB = 2000


def _tc_edge(pg, qg, w2, b2):
    def body(pg_ref, qg_ref, w2_ref, b2_ref, out_ref):
        h = jnp.maximum(pg_ref[...] + qg_ref[...], 0.0)
        out_ref[...] = _dot(h, w2_ref[...]) + b2_ref[...]

    return pl.pallas_call(
        body,
        grid=(E // EB,),
        in_specs=[
            pl.BlockSpec((EB, D), lambda i: (i, 0)),
            pl.BlockSpec((EB, D), lambda i: (i, 0)),
            pl.BlockSpec((D, D), lambda i: (0, 0)),
            pl.BlockSpec((1, D), lambda i: (0, 0)),
        ],
        out_specs=pl.BlockSpec((EB, D), lambda i: (i, 0)),
        out_shape=jax.ShapeDtypeStruct((E, D), _F32),
        compiler_params=pltpu.CompilerParams(
            dimension_semantics=("parallel",)),
    )(pg, qg, w2, b2.reshape(1, D))


def _sc_segsum(ea, col, ztr):
    mesh = plsc.VectorSubcoreMesh(core_axis_name="c", subcore_axis_name="s")

    @functools.partial(
        pl.kernel,
        mesh=mesh,
        out_type=jax.ShapeDtypeStruct((NC * N, D), _F32),
        scratch_types=[
            pltpu.VMEM((PER_TILE,), jnp.int32),
            pltpu.VMEM((CH,), jnp.int32),
            pltpu.VMEM((CH,), jnp.int32),
            pltpu.VMEM((CH, D), _F32),
            pltpu.VMEM_SHARED((TR, D), _F32),
            pltpu.VMEM_SHARED((TR, D), _F32),
            pltpu.SemaphoreType.DMA,
        ],
    )
    def sk(ea_hbm, col_hbm, ztr_hbm, part_hbm,
           idxc_all, idxa, idxb, buf, ta_sh, tb_sh, semr):
        c = lax.axis_index("c")
        s = lax.axis_index("s")

        @pl.when(s == 0)
        def _():
            pltpu.sync_copy(ztr_hbm, ta_sh)
            pltpu.sync_copy(ztr_hbm, tb_sh)

        wid = s * NC + c
        base = wid * PER_TILE
        pltpu.sync_copy(col_hbm.at[pl.ds(base, PER_TILE)], idxc_all)
        plsc.subcore_barrier()

        @pl.loop(0, PER_TILE, step=CH)
        def _(off):
            b = base + off
            cp = pltpu.async_copy(ea_hbm.at[pl.ds(b, CH)], buf, semr)
            _split_idx(idxc_all, idxa, idxb, off)
            cp.wait()
            pltpu.sync_copy(buf, ta_sh.at[idxa], add=True)
            pltpu.sync_copy(buf, tb_sh.at[idxb], add=True)

        plsc.subcore_barrier()

        @pl.when(s == 0)
        def _():
            pltpu.sync_copy(ta_sh.at[pl.ds(0, NH)],
                            part_hbm.at[pl.ds(c * N, NH)])
            pltpu.sync_copy(tb_sh.at[pl.ds(0, NH)],
                            part_hbm.at[pl.ds(c * N + NH, NH)])

    return sk(ea, col, ztr)


def _tc_node(x0, part, cnt, wn1a, wn1b, bn1, wn2, bn2, wg, bg):
    def body(x_ref, part_ref, cnt_ref, wa_ref, wb_ref, b1_ref,
             w2_ref, b2_ref, wg_ref, bg_ref, xo_ref, u_ref):
        cntv = cnt_ref[0, :, 0:1] + cnt_ref[1, :, 0:1]
        agg = (part_ref[0] + part_ref[1]) / jnp.maximum(cntv, 1.0)
        h2 = jnp.maximum(
            _dot(x_ref[...], wa_ref[...]) + _dot(agg, wb_ref[...])
            + b1_ref[...], 0.0)
        xo = _dot(h2, w2_ref[...]) + b2_ref[...]
        xo_ref[...] = xo
        u_ref[...] = _dot(jnp.sum(xo, axis=0, keepdims=True) / N,
                          wg_ref[...]) + bg_ref[...]

    return pl.pallas_call(
        body,
        out_shape=(jax.ShapeDtypeStruct((N, D), _F32),
                   jax.ShapeDtypeStruct((1, D), _F32)),
    )(x0, part, cnt, wn1a, wn1b, bn1.reshape(1, D),
      wn2, bn2.reshape(1, D), wg, bg.reshape(1, D))


def kernel(x, edge_index, W_e1, b_e1, W_e2, b_e2,
           W_n1, b_n1, W_n2, b_n2, W_g, b_g):
    x0 = x[0]
    row = edge_index[0]
    col = edge_index[1]
    ztr = jnp.zeros((TR, D), _F32)
    ones_c = jnp.ones((CH, D), _F32)

    p, q = _tc_pq(x0, W_e1[:D], W_e1[D:], b_e1)
    pg, qg, cnt2 = _sc_gather(p, q, row, col, ztr, ones_c)
    ea = _tc_edge(pg, qg, W_e2, b_e2)
    part2 = _sc_segsum(ea, col, ztr)
    part = part2.reshape(NC, N, D)
    cnt = cnt2.reshape(NC, N, D)
    x_out, u = _tc_node(x0, part, cnt, W_n1[:D], W_n1[D:], b_n1,
                        W_n2, b_n2, W_g, b_g)
    return (x_out[None], ea[None], u)
